# baseline (speedup 1.0000x reference)
import functools

import jax
import jax.numpy as jnp
from jax import lax
from jax.experimental import pallas as pl
from jax.experimental.pallas import tpu as pltpu
from jax.experimental.pallas import tpu_sc as plsc

_B, _C = 16, 96
_NTHETA, _NY, _NX = 8, 64, 64
_ROWS = _B * _C
_IN_ROW = _NTHETA * _NY * _NX
_OUT_ROW = _IN_ROW // 4
_NW = 32
_LINES = _NTHETA * (_NY // 2)

_SC_ROWS = 512
_RPW = _SC_ROWS // _NW
_TC_ROWS = _ROWS - _SC_ROWS
_TCB = 32

_mesh = plsc.VectorSubcoreMesh(core_axis_name="c", subcore_axis_name="s")


_SCB = _SC_ROWS // _TCB


@functools.partial(
    pl.kernel,
    mesh=_mesh,
    out_type=jax.ShapeDtypeStruct((_SC_ROWS, _OUT_ROW), jnp.float32),
    scratch_types=[
        pltpu.VMEM((_IN_ROW,), jnp.float32),
        pltpu.VMEM((_IN_ROW,), jnp.float32),
        pltpu.VMEM((_OUT_ROW,), jnp.float32),
        pltpu.VMEM((_OUT_ROW,), jnp.float32),
        pltpu.SemaphoreType.DMA,
        pltpu.SemaphoreType.DMA,
        pltpu.SemaphoreType.DMA,
        pltpu.SemaphoreType.DMA,
    ],
    compiler_params=pltpu.CompilerParams(needs_layout_passes=False),
)
def _pool_sc(x_hbm, out_hbm, in0, in1, o0, o1, si0, si1, so0, so1):
    wid = lax.axis_index("s") * 2 + lax.axis_index("c")
    row0 = wid * _RPW
    in_v = (in0, in1)
    out_v = (o0, o1)
    sem_i = (si0, si1)
    sem_o = (so0, so1)
    iota = lax.broadcasted_iota(jnp.int32, (16,), 0)
    base_idx = [
        [2 * iota + 32 * g + off for off in (0, 1, 64, 65)] for g in (0, 1)
    ]

    def compute_row(src, dst):
        def line_body(l, _):
            off = l * 128
            for g in (0, 1):
                v = [plsc.load_gather(src, [e + off]) for e in base_idx[g]]
                dst[pl.ds(l * 32 + 16 * g, 16)] = (
                    (v[0] + v[1]) + (v[2] + v[3])
                ) * 0.25
            return 0

        lax.fori_loop(0, _LINES, line_body, 0, unroll=4)

    pltpu.async_copy(x_hbm.at[row0], in0, si0)

    def pair_body(ii, _):
        for b in (0, 1):
            i = 2 * ii + b
            r = row0 + i
            if b == 0:
                pltpu.async_copy(x_hbm.at[r + 1], in_v[1], sem_i[1])
            else:
                @pl.when(ii < _RPW // 2 - 1)
                def _():
                    pltpu.async_copy(x_hbm.at[r + 1], in_v[0], sem_i[0])

            pltpu.make_async_copy(x_hbm.at[row0], in_v[b], sem_i[b]).wait()
            @pl.when(ii >= 1)
            def _():
                pltpu.make_async_copy(
                    out_v[b], out_hbm.at[row0], sem_o[b]
                ).wait()

            compute_row(in_v[b], out_v[b])
            pltpu.async_copy(out_v[b], out_hbm.at[r], sem_o[b])
        return 0

    lax.fori_loop(0, _RPW // 2, pair_body, 0)
    for b in (0, 1):
        pltpu.make_async_copy(out_v[b], out_hbm.at[row0], sem_o[b]).wait()


def _pool_tc_body(x_ref, o_ref):
    v = x_ref[...].reshape(_TCB * _LINES, 128)
    ri = lax.broadcasted_iota(jnp.int32, (128, 32), 0)
    ci = lax.broadcasted_iota(jnp.int32, (128, 32), 1)
    q = jnp.where((ri % 64) // 2 == ci, jnp.float32(0.25), jnp.float32(0.0))
    o_ref[...] = jnp.dot(
        v, q, preferred_element_type=jnp.float32
    ).reshape(_TCB, _LINES, 32)


_pool_tc = pl.pallas_call(
    _pool_tc_body,
    grid=(_TC_ROWS // _TCB,),
    in_specs=[pl.BlockSpec((_TCB, _LINES, 128), lambda i: (i + _SCB, 0, 0))],
    out_specs=pl.BlockSpec((_TCB, _LINES, 32), lambda i: (i + _SCB, 0, 0)),
    out_shape=jax.ShapeDtypeStruct((_ROWS, _LINES, 32), jnp.float32),
)


def _fill_body(full_ref, sc_ref, o_ref):
    del full_ref
    o_ref[...] = sc_ref[...]


_fill = pl.pallas_call(
    _fill_body,
    grid=(_SCB,),
    in_specs=[
        pl.BlockSpec(memory_space=pl.ANY),
        pl.BlockSpec((_TCB, _LINES, 32), lambda i: (i, 0, 0)),
    ],
    out_specs=pl.BlockSpec((_TCB, _LINES, 32), lambda i: (i, 0, 0)),
    out_shape=jax.ShapeDtypeStruct((_ROWS, _LINES, 32), jnp.float32),
    input_output_aliases={0: 0},
)


def kernel(x):
    xr = x.reshape(_ROWS, _IN_ROW)
    out_sc = _pool_sc(xr[:_SC_ROWS])
    full = _pool_tc(xr.reshape(_ROWS, _LINES, 128))
    out = _fill(full, out_sc.reshape(_SC_ROWS, _LINES, 32))
    return out.reshape(_B, _C, _OUT_ROW)


# speedup vs baseline: 1.4840x; 1.4840x over previous
import functools

import jax
import jax.numpy as jnp
from jax import lax
from jax.experimental import pallas as pl
from jax.experimental.pallas import tpu as pltpu
from jax.experimental.pallas import tpu_sc as plsc

_B, _C = 16, 96
_NTHETA, _NY, _NX = 8, 64, 64
_ROWS = _B * _C
_IN_ROW = _NTHETA * _NY * _NX
_OUT_ROW = _IN_ROW // 4
_NW = 32
_LINES = _NTHETA * (_NY // 2)

_SC_ROWS = 512
_RPW = _SC_ROWS // _NW
_TC_ROWS = _ROWS - _SC_ROWS
_TCB = 32

_mesh = plsc.VectorSubcoreMesh(core_axis_name="c", subcore_axis_name="s")


_SCB = _SC_ROWS // _TCB


@functools.partial(
    pl.kernel,
    mesh=_mesh,
    out_type=jax.ShapeDtypeStruct((_SC_ROWS, _OUT_ROW), jnp.float32),
    scratch_types=[
        pltpu.VMEM((_IN_ROW,), jnp.float32),
        pltpu.VMEM((_IN_ROW,), jnp.float32),
        pltpu.VMEM((_OUT_ROW,), jnp.float32),
        pltpu.VMEM((_OUT_ROW,), jnp.float32),
        pltpu.SemaphoreType.DMA,
        pltpu.SemaphoreType.DMA,
        pltpu.SemaphoreType.DMA,
        pltpu.SemaphoreType.DMA,
    ],
    compiler_params=pltpu.CompilerParams(needs_layout_passes=False),
)
def _pool_sc(x_hbm, out_hbm, in0, in1, o0, o1, si0, si1, so0, so1):
    wid = lax.axis_index("s") * 2 + lax.axis_index("c")
    row0 = wid * _RPW
    in_v = (in0, in1)
    out_v = (o0, o1)
    sem_i = (si0, si1)
    sem_o = (so0, so1)
    iota = lax.broadcasted_iota(jnp.int32, (16,), 0)
    base_idx = [
        [2 * iota + 32 * g + off for off in (0, 1, 64, 65)] for g in (0, 1)
    ]

    def compute_row(src, dst):
        def line_body(l, _):
            off = l * 128
            for g in (0, 1):
                v = [plsc.load_gather(src, [e + off]) for e in base_idx[g]]
                dst[pl.ds(l * 32 + 16 * g, 16)] = (
                    (v[0] + v[1]) + (v[2] + v[3])
                ) * 0.25
            return 0

        lax.fori_loop(0, _LINES, line_body, 0, unroll=4)

    pltpu.async_copy(x_hbm.at[row0], in0, si0)

    def pair_body(ii, _):
        for b in (0, 1):
            i = 2 * ii + b
            r = row0 + i
            if b == 0:
                pltpu.async_copy(x_hbm.at[r + 1], in_v[1], sem_i[1])
            else:
                @pl.when(ii < _RPW // 2 - 1)
                def _():
                    pltpu.async_copy(x_hbm.at[r + 1], in_v[0], sem_i[0])

            pltpu.make_async_copy(x_hbm.at[row0], in_v[b], sem_i[b]).wait()
            @pl.when(ii >= 1)
            def _():
                pltpu.make_async_copy(
                    out_v[b], out_hbm.at[row0], sem_o[b]
                ).wait()

            compute_row(in_v[b], out_v[b])
            pltpu.async_copy(out_v[b], out_hbm.at[r], sem_o[b])
        return 0

    lax.fori_loop(0, _RPW // 2, pair_body, 0)
    for b in (0, 1):
        pltpu.make_async_copy(out_v[b], out_hbm.at[row0], sem_o[b]).wait()


def _pool_tc_body(x_ref, o_ref):
    v = x_ref[...].reshape(_TCB * (_LINES // 4), 512)
    ri = lax.broadcasted_iota(jnp.int32, (512, 128), 0)
    ci = lax.broadcasted_iota(jnp.int32, (512, 128), 1)
    q = jnp.where(
        (ri // 128) * 32 + ((ri % 128) % 64) // 2 == ci,
        jnp.float32(0.25),
        jnp.float32(0.0),
    )
    o_ref[...] = jnp.dot(
        v, q, preferred_element_type=jnp.float32
    ).reshape(_TCB, _LINES // 4, 128)


_pool_tc = pl.pallas_call(
    _pool_tc_body,
    grid=(_TC_ROWS // _TCB,),
    in_specs=[pl.BlockSpec((_TCB, _LINES, 128), lambda i: (i + _SCB, 0, 0))],
    out_specs=pl.BlockSpec(
        (_TCB, _LINES // 4, 128), lambda i: (i + _SCB, 0, 0)
    ),
    out_shape=jax.ShapeDtypeStruct((_ROWS, _LINES // 4, 128), jnp.float32),
)


def _fill_body(full_ref, sc_ref, o_ref):
    del full_ref
    o_ref[...] = sc_ref[...]


_fill = pl.pallas_call(
    _fill_body,
    grid=(_SCB,),
    in_specs=[
        pl.BlockSpec(memory_space=pl.ANY),
        pl.BlockSpec((_TCB, _LINES // 4, 128), lambda i: (i, 0, 0)),
    ],
    out_specs=pl.BlockSpec((_TCB, _LINES // 4, 128), lambda i: (i, 0, 0)),
    out_shape=jax.ShapeDtypeStruct((_ROWS, _LINES // 4, 128), jnp.float32),
    input_output_aliases={0: 0},
)


def kernel(x):
    xr = x.reshape(_ROWS, _IN_ROW)
    out_sc = _pool_sc(xr)
    full = _pool_tc(xr.reshape(_ROWS, _LINES, 128))
    out = _fill(full, out_sc.reshape(_SC_ROWS, _LINES // 4, 128))
    return out.reshape(_B, _C, _OUT_ROW)


# speedup vs baseline: 2.2635x; 1.5253x over previous
import functools

import jax
import jax.numpy as jnp
from jax import lax
from jax.experimental import pallas as pl
from jax.experimental.pallas import tpu as pltpu
from jax.experimental.pallas import tpu_sc as plsc

_B, _C = 16, 96
_NTHETA, _NY, _NX = 8, 64, 64
_ROWS = _B * _C
_IN_ROW = _NTHETA * _NY * _NX
_OUT_ROW = _IN_ROW // 4
_NW = 32
_RPW = _ROWS // _NW
_LINES = _NTHETA * (_NY // 2)

_mesh = plsc.VectorSubcoreMesh(core_axis_name="c", subcore_axis_name="s")


@functools.partial(
    pl.kernel,
    mesh=_mesh,
    out_type=jax.ShapeDtypeStruct((_ROWS, _OUT_ROW), jnp.float32),
    scratch_types=[
        pltpu.VMEM((_IN_ROW,), jnp.float32),
        pltpu.VMEM((_IN_ROW,), jnp.float32),
        pltpu.VMEM((_OUT_ROW,), jnp.float32),
        pltpu.VMEM((_OUT_ROW,), jnp.float32),
        pltpu.SemaphoreType.DMA,
        pltpu.SemaphoreType.DMA,
        pltpu.SemaphoreType.DMA,
        pltpu.SemaphoreType.DMA,
    ],
    compiler_params=pltpu.CompilerParams(needs_layout_passes=False),
)
def _pool_sc(x_hbm, out_hbm, in0, in1, o0, o1, si0, si1, so0, so1):
    wid = lax.axis_index("s") * 2 + lax.axis_index("c")
    row0 = wid * _RPW
    in_v = (in0, in1)
    out_v = (o0, o1)
    sem_i = (si0, si1)
    sem_o = (so0, so1)
    iota = lax.broadcasted_iota(jnp.int32, (16,), 0)
    idx = [2 * iota + off for off in (0, 1, 64, 65)]

    def compute_row(src, dst):
        def line_body(l, _):
            for g in (0, 1):
                win = src.at[pl.ds(l * 128 + 32 * g, 96)]
                v = [plsc.load_gather(win, [e]) for e in idx]
                dst[pl.ds(l * 32 + 16 * g, 16)] = (
                    (v[0] + v[1]) + (v[2] + v[3])
                ) * 0.25
            return 0

        lax.fori_loop(0, _LINES, line_body, 0, unroll=4)

    pltpu.async_copy(x_hbm.at[row0], in0, si0)

    def pair_body(ii, _):
        for b in (0, 1):
            i = 2 * ii + b
            r = row0 + i
            if b == 0:
                pltpu.async_copy(x_hbm.at[r + 1], in_v[1], sem_i[1])
            else:
                @pl.when(ii < _RPW // 2 - 1)
                def _():
                    pltpu.async_copy(x_hbm.at[r + 1], in_v[0], sem_i[0])

            pltpu.make_async_copy(x_hbm.at[row0], in_v[b], sem_i[b]).wait()
            @pl.when(ii >= 1)
            def _():
                pltpu.make_async_copy(
                    out_v[b], out_hbm.at[row0], sem_o[b]
                ).wait()

            compute_row(in_v[b], out_v[b])
            pltpu.async_copy(out_v[b], out_hbm.at[r], sem_o[b])
        return 0

    lax.fori_loop(0, _RPW // 2, pair_body, 0)
    for b in (0, 1):
        pltpu.make_async_copy(out_v[b], out_hbm.at[row0], sem_o[b]).wait()


def kernel(x):
    out = _pool_sc(x.reshape(_ROWS, _IN_ROW))
    return out.reshape(_B, _C, _OUT_ROW)


# speedup vs baseline: 2.2717x; 1.0036x over previous
import functools

import jax
import jax.numpy as jnp
from jax import lax
from jax.experimental import pallas as pl
from jax.experimental.pallas import tpu as pltpu
from jax.experimental.pallas import tpu_sc as plsc

_B, _C = 16, 96
_NTHETA, _NY, _NX = 8, 64, 64
_ROWS = _B * _C
_IN_ROW = _NTHETA * _NY * _NX
_OUT_ROW = _IN_ROW // 4
_NW = 32
_RPW = _ROWS // _NW
_LINES = _NTHETA * (_NY // 2)

_mesh = plsc.VectorSubcoreMesh(core_axis_name="c", subcore_axis_name="s")


@functools.partial(
    pl.kernel,
    mesh=_mesh,
    out_type=jax.ShapeDtypeStruct((_ROWS, _OUT_ROW), jnp.float32),
    scratch_types=[
        pltpu.VMEM((_IN_ROW,), jnp.float32),
        pltpu.VMEM((_IN_ROW,), jnp.float32),
        pltpu.VMEM((_OUT_ROW,), jnp.float32),
        pltpu.VMEM((_OUT_ROW,), jnp.float32),
        pltpu.SemaphoreType.DMA,
        pltpu.SemaphoreType.DMA,
        pltpu.SemaphoreType.DMA,
        pltpu.SemaphoreType.DMA,
    ],
    compiler_params=pltpu.CompilerParams(needs_layout_passes=False),
)
def _pool_sc(x_hbm, out_hbm, in0, in1, o0, o1, si0, si1, so0, so1):
    wid = lax.axis_index("s") * 2 + lax.axis_index("c")
    row0 = wid * _RPW
    in_v = (in0, in1)
    out_v = (o0, o1)
    sem_i = (si0, si1)
    sem_o = (so0, so1)
    iota = lax.broadcasted_iota(jnp.int32, (16,), 0)
    idx = [2 * iota + off for off in (0, 1, 64, 65)]

    def compute_row(src, dst):
        def line_body(l, _):
            for g in (0, 1):
                win = src.at[pl.ds(l * 128 + 32 * g, 96)]
                v = [plsc.load_gather(win, [e]) for e in idx]
                dst[pl.ds(l * 32 + 16 * g, 16)] = (
                    (v[0] + v[1]) + (v[2] + v[3])
                ) * 0.25
            return 0

        lax.fori_loop(0, _LINES, line_body, 0, unroll=8)

    pltpu.async_copy(x_hbm.at[row0], in0, si0)

    def pair_body(ii, _):
        for b in (0, 1):
            i = 2 * ii + b
            r = row0 + i
            if b == 0:
                pltpu.async_copy(x_hbm.at[r + 1], in_v[1], sem_i[1])
            else:
                @pl.when(ii < _RPW // 2 - 1)
                def _():
                    pltpu.async_copy(x_hbm.at[r + 1], in_v[0], sem_i[0])

            pltpu.make_async_copy(x_hbm.at[row0], in_v[b], sem_i[b]).wait()
            @pl.when(ii >= 1)
            def _():
                pltpu.make_async_copy(
                    out_v[b], out_hbm.at[row0], sem_o[b]
                ).wait()

            compute_row(in_v[b], out_v[b])
            pltpu.async_copy(out_v[b], out_hbm.at[r], sem_o[b])
        return 0

    lax.fori_loop(0, _RPW // 2, pair_body, 0)
    for b in (0, 1):
        pltpu.make_async_copy(out_v[b], out_hbm.at[row0], sem_o[b]).wait()


def kernel(x):
    out = _pool_sc(x.reshape(_ROWS, _IN_ROW))
    return out.reshape(_B, _C, _OUT_ROW)
